# Initial kernel scaffold; baseline (speedup 1.0000x reference)
import jax
import jax.numpy as jnp
from jax.experimental import pallas as pl


def kernel(x, W_gate, W_noise):
    raise NotImplementedError("write your pallas kernel here")


# speedup vs baseline: 1.3537x; 1.3537x over previous
import jax
import jax.numpy as jnp
from jax.experimental import pallas as pl

D_MODEL = 4096
NUM_EXPERTS = 64
TOP_K = 8


def _gating_kernel(x_ref, w_ref, gates_ref, idx_ref):
    x = x_ref[...]
    w = w_ref[...]
    logits = jax.lax.dot_general(
        x, w, (((1,), (1,)), ((), ())), preferred_element_type=jnp.float32)
    iota = jax.lax.broadcasted_iota(jnp.int32, logits.shape, 1)
    work = logits
    vals, idxs = [], []
    for _ in range(TOP_K):
        m = jnp.max(work, axis=1, keepdims=True)
        hit = jnp.min(jnp.where(work == m, iota, NUM_EXPERTS),
                      axis=1, keepdims=True)
        vals.append(m)
        idxs.append(hit)
        work = jnp.where(iota == hit, -jnp.inf, work)
    v = jnp.concatenate(vals, axis=1)
    ix = jnp.concatenate(idxs, axis=1)
    e = jnp.exp(v - v[:, :1])
    gates_ref[...] = e / jnp.sum(e, axis=1, keepdims=True)
    idx_ref[...] = ix


def kernel(x, W_gate, W_noise):
    B, N, D = x.shape
    T = B * N
    xf = x.reshape(T, D)
    BT = 512
    gates, idx = pl.pallas_call(
        _gating_kernel,
        grid=(T // BT,),
        in_specs=[
            pl.BlockSpec((BT, D), lambda i: (i, 0)),
            pl.BlockSpec((NUM_EXPERTS, D), lambda i: (0, 0)),
        ],
        out_specs=[
            pl.BlockSpec((BT, TOP_K), lambda i: (i, 0)),
            pl.BlockSpec((BT, TOP_K), lambda i: (i, 0)),
        ],
        out_shape=[
            jax.ShapeDtypeStruct((T, TOP_K), jnp.float32),
            jax.ShapeDtypeStruct((T, TOP_K), jnp.int32),
        ],
    )(xf, W_gate)
    return gates.reshape(B, N, TOP_K), idx.reshape(B, N, TOP_K)


# speedup vs baseline: 1.3630x; 1.0069x over previous
import jax
import jax.numpy as jnp
from jax.experimental import pallas as pl
from jax.experimental.pallas import tpu as pltpu

D_MODEL = 4096
NUM_EXPERTS = 64
TOP_K = 8


def _gating_kernel(x_ref, w_ref, gates_ref, idx_ref):
    x = x_ref[...]
    w = w_ref[...]
    logits = jax.lax.dot_general(
        x, w, (((1,), (1,)), ((), ())), preferred_element_type=jnp.float32)
    iota = jax.lax.broadcasted_iota(jnp.int32, logits.shape, 1)
    work = logits
    vals, idxs = [], []
    for _ in range(TOP_K):
        m = jnp.max(work, axis=1, keepdims=True)
        hit = jnp.min(jnp.where(work == m, iota, NUM_EXPERTS),
                      axis=1, keepdims=True)
        vals.append(m)
        idxs.append(hit)
        work = jnp.where(iota == hit, -jnp.inf, work)
    v = jnp.concatenate(vals, axis=1)
    ix = jnp.concatenate(idxs, axis=1)
    e = jnp.exp(v - v[:, :1])
    gates_ref[...] = e / jnp.sum(e, axis=1, keepdims=True)
    idx_ref[...] = ix


def kernel(x, W_gate, W_noise):
    B, N, D = x.shape
    T = B * N
    xf = x.reshape(T, D)
    BT = 512
    gates, idx = pl.pallas_call(
        _gating_kernel,
        grid=(T // BT,),
        in_specs=[
            pl.BlockSpec((BT, D), lambda i: (i, 0)),
            pl.BlockSpec((NUM_EXPERTS, D), lambda i: (0, 0)),
        ],
        out_specs=[
            pl.BlockSpec((BT, TOP_K), lambda i: (i, 0)),
            pl.BlockSpec((BT, TOP_K), lambda i: (i, 0)),
        ],
        out_shape=[
            jax.ShapeDtypeStruct((T, TOP_K), jnp.float32),
            jax.ShapeDtypeStruct((T, TOP_K), jnp.int32),
        ],
        compiler_params=pltpu.CompilerParams(
            dimension_semantics=("parallel",)),
    )(xf, W_gate)
    return gates.reshape(B, N, TOP_K), idx.reshape(B, N, TOP_K)


# speedup vs baseline: 2.2615x; 1.6592x over previous
import jax
import jax.numpy as jnp
from jax.experimental import pallas as pl
from jax.experimental.pallas import tpu as pltpu

D_MODEL = 4096
NUM_EXPERTS = 64
TOP_K = 8


def _gating_kernel(x_ref, w_ref, gates_ref, idx_ref):
    x = x_ref[...]
    w = w_ref[...]
    logits = jax.lax.dot_general(
        w, x, (((1,), (1,)), ((), ())),
        preferred_element_type=jnp.float32)
    iota = jax.lax.broadcasted_iota(jnp.int32, logits.shape, 0)
    work = logits
    vals, idxs = [], []
    for _ in range(TOP_K):
        m = jnp.max(work, axis=0, keepdims=True)
        hit = jnp.min(jnp.where(work == m, iota, NUM_EXPERTS),
                      axis=0, keepdims=True)
        vals.append(m)
        idxs.append(hit)
        work = jnp.where(iota == hit, -jnp.inf, work)
    v = jnp.concatenate(vals, axis=0)
    ix = jnp.concatenate(idxs, axis=0)
    e = jnp.exp(v - v[:1])
    gates_ref[...] = e / jnp.sum(e, axis=0, keepdims=True)
    idx_ref[...] = ix


def kernel(x, W_gate, W_noise):
    B, N, D = x.shape
    T = B * N
    xf = x.reshape(T, D)
    BT = 512
    gates_t, idx_t = pl.pallas_call(
        _gating_kernel,
        grid=(T // BT,),
        in_specs=[
            pl.BlockSpec((BT, D), lambda i: (i, 0)),
            pl.BlockSpec((NUM_EXPERTS, D), lambda i: (0, 0)),
        ],
        out_specs=[
            pl.BlockSpec((TOP_K, BT), lambda i: (0, i)),
            pl.BlockSpec((TOP_K, BT), lambda i: (0, i)),
        ],
        out_shape=[
            jax.ShapeDtypeStruct((TOP_K, T), jnp.float32),
            jax.ShapeDtypeStruct((TOP_K, T), jnp.int32),
        ],
        compiler_params=pltpu.CompilerParams(
            dimension_semantics=("parallel",)),
    )(xf, W_gate)
    gates = gates_t.T.reshape(B, N, TOP_K)
    idx = idx_t.T.reshape(B, N, TOP_K)
    return gates, idx


# speedup vs baseline: 2.3615x; 1.0442x over previous
import jax
import jax.numpy as jnp
from jax.experimental import pallas as pl
from jax.experimental.pallas import tpu as pltpu

D_MODEL = 4096
NUM_EXPERTS = 64
TOP_K = 8


def _gating_kernel(x_ref, w_ref, gates_ref, idx_ref):
    x = x_ref[...]
    w = w_ref[...]
    logits = jax.lax.dot_general(
        w, x, (((1,), (1,)), ((), ())),
        preferred_element_type=jnp.float32)
    iota = jax.lax.broadcasted_iota(jnp.int32, logits.shape, 0)
    work = logits
    vals, idxs = [], []
    for _ in range(TOP_K):
        m = jnp.max(work, axis=0, keepdims=True)
        hit = jnp.min(jnp.where(work == m, iota, NUM_EXPERTS),
                      axis=0, keepdims=True)
        vals.append(m)
        idxs.append(hit)
        work = jnp.where(iota == hit, -jnp.inf, work)
    v = jnp.concatenate(vals, axis=0)
    ix = jnp.concatenate(idxs, axis=0)
    e = jnp.exp(v - v[:1])
    gates_ref[...] = e / jnp.sum(e, axis=0, keepdims=True)
    idx_ref[...] = ix


def kernel(x, W_gate, W_noise):
    B, N, D = x.shape
    T = B * N
    xf = x.reshape(T, D)
    BT = 1024
    gates_t, idx_t = pl.pallas_call(
        _gating_kernel,
        grid=(T // BT,),
        in_specs=[
            pl.BlockSpec((BT, D), lambda i: (i, 0)),
            pl.BlockSpec((NUM_EXPERTS, D), lambda i: (0, 0)),
        ],
        out_specs=[
            pl.BlockSpec((TOP_K, BT), lambda i: (0, i)),
            pl.BlockSpec((TOP_K, BT), lambda i: (0, i)),
        ],
        out_shape=[
            jax.ShapeDtypeStruct((TOP_K, T), jnp.float32),
            jax.ShapeDtypeStruct((TOP_K, T), jnp.int32),
        ],
        compiler_params=pltpu.CompilerParams(
            dimension_semantics=("parallel",)),
    )(xf, W_gate)
    gates = gates_t.T.reshape(B, N, TOP_K)
    idx = idx_t.T.reshape(B, N, TOP_K)
    return gates, idx
